# baseline (speedup 1.0000x reference)
import jax
import jax.numpy as jnp
from jax.experimental import pallas as pl
from jax.experimental.pallas import tpu as pltpu

EPS = 1e-06
D = 1024
NH = 16
NKV = 4
HD = 64
RD = 32
E = 8
TOPK = 2
NG = 4
TKG = 2
RSF = 1.0
FF = 512
S = 2048

BT = 256
BQ = 512
BK = 256
GQ = NH // NKV
QKVW = NH * HD + 2 * NKV * HD
FH = FF // 2
W = GQ * BQ

_bf16 = jnp.bfloat16
_f32 = jnp.float32


def _dot(a, b, dims):
    return jax.lax.dot_general(a, b, (dims, ((), ())),
                               preferred_element_type=_f32)


def _qkv_body(x_ref, w_ref, ln_ref, qkvt_ref, xt_ref):
    x = x_ref[...]
    xt_ref[...] = x.T
    var = jnp.mean(x * x, axis=-1, keepdims=True)
    h = x * jax.lax.rsqrt(var + EPS) * ln_ref[...]
    qkv = _dot(h.astype(_bf16), w_ref[...], ((1,), (0,)))
    qkvt_ref[...] = qkv.T


def _norm_rope_t(xt, nw, ct, st, scale):
    var = jnp.mean(xt * xt, axis=0, keepdims=True)
    xt = xt * jax.lax.rsqrt(var + EPS) * (nw * scale)
    x1 = xt[:RD // 2, :]
    x2 = xt[RD // 2:RD, :]
    xp = xt[RD:, :]
    r1 = x1 * ct[:RD // 2, :] - x2 * st[:RD // 2, :]
    r2 = x2 * ct[RD // 2:, :] + x1 * st[RD // 2:, :]
    return jnp.concatenate([r1, r2, xp], axis=0)


VROWS = HD + 8


def _attn_body(q_ref, k_ref, v_ref, cos_ref, sin_ref, qn_ref, kn_ref,
               o_ref, kb_ref, vb_ref):
    i = pl.program_id(1)

    @pl.when(i == 0)
    def _():
        kt = _norm_rope_t(k_ref[0], kn_ref[...], cos_ref[...], sin_ref[...],
                          1.0)
        kb_ref[...] = kt.T.astype(_bf16)
        vb_ref[:HD, :] = v_ref[0].astype(_bf16)
        vb_ref[HD:HD + 1, :] = jnp.ones((1, S), _bf16)
        vb_ref[HD + 1:, :] = jnp.zeros((VROWS - HD - 1, S), _bf16)

    qblk = q_ref[0]
    qs = jnp.concatenate([qblk[h * HD:(h + 1) * HD, :] for h in range(GQ)],
                         axis=1)
    ct = cos_ref[:, pl.ds(i * BQ, BQ)]
    st = sin_ref[:, pl.ds(i * BQ, BQ)]
    ct4 = jnp.concatenate([ct] * GQ, axis=1)
    st4 = jnp.concatenate([st] * GQ, axis=1)
    qt = _norm_rope_t(qs, qn_ref[...], ct4, st4, HD ** -0.5).astype(_bf16)

    def tile(j, masked):
        kc = kb_ref[pl.ds(j * BK, BK), :]
        s = _dot(kc, qt, ((1,), (0,)))
        p = jnp.exp(s)
        if masked:
            kpos = j * BK + jax.lax.broadcasted_iota(jnp.int32, (BK, W), 0)
            qpos = i * BQ + (jax.lax.broadcasted_iota(jnp.int32, (BK, W), 1)
                             & (BQ - 1))
            p = jnp.where(kpos <= qpos, p, 0.0)
        return _dot(vb_ref[:, pl.ds(j * BK, BK)], p.astype(_bf16),
                    ((1,), (0,)))

    def pair(t, acc, masked):
        return acc + tile(2 * t, masked) + tile(2 * t + 1, masked)

    acc = jnp.zeros((VROWS, W), _f32)
    acc = jax.lax.fori_loop(0, i, lambda t, a: pair(t, a, False), acc)
    acc = pair(i, acc, True)
    o_ref[0] = acc[:HD, :] / acc[HD:HD + 1, :]


def _router_body(a0_ref, a1_ref, a2_ref, a3_ref, owb_ref, xt_ref, ln2_ref,
                 ghi_ref, glo_ref, hid_ref, flat_ref, comb_ref):
    h2 = xt_ref[...]
    for g, aref in enumerate((a0_ref, a1_ref, a2_ref, a3_ref)):
        h2 = h2 + _dot(owb_ref[g], aref[...].astype(_bf16), ((1,), (0,)))
    hid_ref[...] = h2
    var = jnp.mean(h2 * h2, axis=0, keepdims=True)
    flat = h2 * jax.lax.rsqrt(var + EPS) * ln2_ref[...]
    flat_ref[...] = flat
    hi = flat.astype(_bf16)
    lo = (flat - hi.astype(_f32)).astype(_bf16)
    logits = (_dot(ghi_ref[...], hi, ((1,), (0,)))
              + _dot(glo_ref[...], hi, ((1,), (0,)))
              + _dot(ghi_ref[...], lo, ((1,), (0,))))
    sc = jax.nn.sigmoid(logits)
    cols = [sc[e:e + 1, :] for e in range(E)]
    gsum = [cols[2 * g] + cols[2 * g + 1] for g in range(NG)]
    neg = jnp.float32(-jnp.inf)
    grank = []
    for gi in range(NG):
        r = jnp.zeros_like(gsum[gi])
        for gj in range(NG):
            if gj == gi:
                continue
            gt = gsum[gj] > gsum[gi]
            if gj < gi:
                gt = jnp.logical_or(gt, gsum[gj] == gsum[gi])
            r = r + gt.astype(_f32)
        grank.append(r)
    masked = [jnp.where(grank[e // 2] < TKG, cols[e], neg) for e in range(E)]
    w = []
    for ei in range(E):
        r = jnp.zeros_like(masked[ei])
        for ej in range(E):
            if ej == ei:
                continue
            gt = masked[ej] > masked[ei]
            if ej < ei:
                gt = jnp.logical_or(gt, masked[ej] == masked[ei])
            r = r + gt.astype(_f32)
        w.append(jnp.where(r < TOPK, cols[ei], 0.0))
    denom = w[0] + w[1] + w[2] + w[3] + w[4] + w[5] + w[6] + w[7] + 1e-20
    comb_ref[...] = jnp.concatenate(w, axis=0) / denom * RSF


def _moe_body(x_ref, h_ref, comb_ref, wg_ref, wu_ref, wd_ref,
              sg_ref, su_ref, sd_ref, out_ref, acc_ref):
    fc = pl.program_id(0)
    t = pl.program_id(1)
    xb = x_ref[...].astype(_bf16)
    tsl = pl.ds(t * BT, BT)

    @pl.when(fc == 0)
    def _():
        acc_ref[:, tsl] = h_ref[...]

    acc = acc_ref[:, tsl]
    acts = []
    for e in range(E):
        g = _dot(wg_ref[e], xb, ((1,), (0,)))
        u = _dot(wu_ref[e], xb, ((1,), (0,)))
        acts.append((g * jax.nn.sigmoid(g) * u).astype(_bf16))
    g = _dot(sg_ref[...], xb, ((1,), (0,)))
    u = _dot(su_ref[...], xb, ((1,), (0,)))
    acts.append((g * jax.nn.sigmoid(g) * u).astype(_bf16))
    for e in range(E):
        y = _dot(wd_ref[e], acts[e], ((1,), (0,)))
        acc = acc + y * comb_ref[e:e + 1, :]
    acc = acc + _dot(sd_ref[...], acts[E], ((1,), (0,)))
    acc_ref[:, tsl] = acc
    out_ref[...] = acc.T


def kernel(hidden_states, cos, sin, ln1_w, ln2_w, q_w, k_w, v_w, o_w,
           qn_w, kn_w, gate_w, ew_gate, ew_up, ew_down,
           sw_gate, sw_up, sw_down):
    x = hidden_states.reshape(S, D)
    cost = cos.reshape(S, RD).T
    sint = sin.reshape(S, RD).T

    wqkv = jnp.concatenate([q_w, k_w, v_w], axis=0).T.astype(_bf16)

    qkvt, xt = pl.pallas_call(
        _qkv_body,
        grid=(S // BT,),
        in_specs=[
            pl.BlockSpec((BT, D), lambda i: (i, 0)),
            pl.BlockSpec((D, QKVW), lambda i: (0, 0)),
            pl.BlockSpec((1, D), lambda i: (0, 0)),
        ],
        out_specs=[
            pl.BlockSpec((QKVW, BT), lambda i: (0, i)),
            pl.BlockSpec((D, BT), lambda i: (0, i)),
        ],
        out_shape=[
            jax.ShapeDtypeStruct((QKVW, S), _f32),
            jax.ShapeDtypeStruct((D, S), _f32),
        ],
        compiler_params=pltpu.CompilerParams(
            dimension_semantics=("parallel",)),
    )(x, wqkv, ln1_w.reshape(1, D))

    q4 = qkvt[:NH * HD].reshape(NKV, GQ * HD, S)
    k3t = qkvt[NH * HD:NH * HD + NKV * HD].reshape(NKV, HD, S)
    v3t = qkvt[NH * HD + NKV * HD:].reshape(NKV, HD, S)

    aa = pl.pallas_call(
        _attn_body,
        grid=(NKV, S // BQ),
        in_specs=[
            pl.BlockSpec((1, GQ * HD, BQ), lambda kv, i: (kv, 0, i)),
            pl.BlockSpec((1, HD, S), lambda kv, i: (kv, 0, 0)),
            pl.BlockSpec((1, HD, S), lambda kv, i: (kv, 0, 0)),
            pl.BlockSpec((RD, S), lambda kv, i: (0, 0)),
            pl.BlockSpec((RD, S), lambda kv, i: (0, 0)),
            pl.BlockSpec((HD, 1), lambda kv, i: (0, 0)),
            pl.BlockSpec((HD, 1), lambda kv, i: (0, 0)),
        ],
        out_specs=pl.BlockSpec((1, HD, W), lambda kv, i: (kv, 0, i)),
        out_shape=jax.ShapeDtypeStruct((NKV, HD, (S // BQ) * W), _f32),
        scratch_shapes=[pltpu.VMEM((S, HD), _bf16),
                        pltpu.VMEM((VROWS, S), _bf16)],
        compiler_params=pltpu.CompilerParams(
            dimension_semantics=("parallel", "arbitrary")),
    )(q4, k3t, v3t, cost, sint, qn_w.reshape(HD, 1), kn_w.reshape(HD, 1))

    aa2 = aa.reshape(NKV * HD, (S // BQ) * W)

    owb = (o_w.reshape(D, NKV, GQ, HD).transpose(2, 0, 1, 3)
           .reshape(GQ, D, NKV * HD).astype(_bf16))
    ghi = gate_w.astype(_bf16)
    glo = (gate_w - ghi.astype(_f32)).astype(_bf16)

    nb = BQ // BT

    def band_spec(g):
        return pl.BlockSpec(
            (NKV * HD, BT),
            lambda t, g=g: (0, (t // nb) * (W // BT) + g * (BQ // BT)
                            + (t % nb)))

    hidt, flatt, combt = pl.pallas_call(
        _router_body,
        grid=(S // BT,),
        in_specs=[
            band_spec(0), band_spec(1), band_spec(2), band_spec(3),
            pl.BlockSpec((GQ, D, NKV * HD), lambda t: (0, 0, 0)),
            pl.BlockSpec((D, BT), lambda t: (0, t)),
            pl.BlockSpec((D, 1), lambda t: (0, 0)),
            pl.BlockSpec((E, D), lambda t: (0, 0)),
            pl.BlockSpec((E, D), lambda t: (0, 0)),
        ],
        out_specs=[
            pl.BlockSpec((D, BT), lambda t: (0, t)),
            pl.BlockSpec((D, BT), lambda t: (0, t)),
            pl.BlockSpec((E, BT), lambda t: (0, t)),
        ],
        out_shape=[
            jax.ShapeDtypeStruct((D, S), _f32),
            jax.ShapeDtypeStruct((D, S), _f32),
            jax.ShapeDtypeStruct((E, S), _f32),
        ],
        compiler_params=pltpu.CompilerParams(
            dimension_semantics=("parallel",)),
    )(aa2, aa2, aa2, aa2, owb, xt, ln2_w.reshape(D, 1), ghi, glo)

    wgb = ew_gate.astype(_bf16)
    wub = ew_up.astype(_bf16)
    wdb = ew_down.astype(_bf16)
    sgb = sw_gate.astype(_bf16)
    sub = sw_up.astype(_bf16)
    sdb = sw_down.astype(_bf16)

    out = pl.pallas_call(
        _moe_body,
        grid=(FF // FH, S // BT),
        in_specs=[
            pl.BlockSpec((D, BT), lambda fc, t: (0, t)),
            pl.BlockSpec((D, BT), lambda fc, t: (0, t)),
            pl.BlockSpec((E, BT), lambda fc, t: (0, t)),
            pl.BlockSpec((E, FH, D), lambda fc, t: (0, fc, 0)),
            pl.BlockSpec((E, FH, D), lambda fc, t: (0, fc, 0)),
            pl.BlockSpec((E, D, FH), lambda fc, t: (0, 0, fc)),
            pl.BlockSpec((FH, D), lambda fc, t: (fc, 0)),
            pl.BlockSpec((FH, D), lambda fc, t: (fc, 0)),
            pl.BlockSpec((D, FH), lambda fc, t: (0, fc)),
        ],
        out_specs=pl.BlockSpec((BT, D), lambda fc, t: (t, 0)),
        out_shape=jax.ShapeDtypeStruct((S, D), _f32),
        scratch_shapes=[pltpu.VMEM((D, S), _f32)],
        compiler_params=pltpu.CompilerParams(
            dimension_semantics=("arbitrary", "parallel")),
    )(flatt, hidt, combt, wgb, wub, wdb, sgb, sub, sdb)

    return out.reshape(1, S, D)


# speedup vs baseline: 1.0438x; 1.0438x over previous
import jax
import jax.numpy as jnp
from jax.experimental import pallas as pl
from jax.experimental.pallas import tpu as pltpu

EPS = 1e-06
D = 1024
NH = 16
NKV = 4
HD = 64
RD = 32
E = 8
TOPK = 2
NG = 4
TKG = 2
RSF = 1.0
FF = 512
S = 2048

BT = 256
BQ = 512
BK = 256
GQ = NH // NKV
QKVW = NH * HD + 2 * NKV * HD
FH = FF // 2
W = GQ * BQ

_bf16 = jnp.bfloat16
_f32 = jnp.float32


def _dot(a, b, dims):
    return jax.lax.dot_general(a, b, (dims, ((), ())),
                               preferred_element_type=_f32)


def _qkv_body(x_ref, w_ref, ln_ref, ct_ref, st_ref, qn_ref, kn_ref,
              qkvt_ref, xt_ref):
    x = x_ref[...]
    xt_ref[...] = x.T
    var = jnp.mean(x * x, axis=-1, keepdims=True)
    h = x * jax.lax.rsqrt(var + EPS) * ln_ref[...]
    qkv = _dot(h.astype(_bf16), w_ref[...], ((1,), (0,)))
    qt = qkv.T
    ct = ct_ref[...]
    st = st_ref[...]
    bands = []
    for hh in range(NH):
        bands.append(_norm_rope_t(qt[hh * HD:(hh + 1) * HD, :], qn_ref[...],
                                  ct, st, HD ** -0.5).astype(_bf16))
    for kk in range(NKV):
        r0 = NH * HD + kk * HD
        bands.append(_norm_rope_t(qt[r0:r0 + HD, :], kn_ref[...],
                                  ct, st, 1.0).astype(_bf16))
    bands.append(qt[NH * HD + NKV * HD:, :].astype(_bf16))
    qkvt_ref[...] = jnp.concatenate(bands, axis=0)


def _norm_rope_t(xt, nw, ct, st, scale):
    var = jnp.mean(xt * xt, axis=0, keepdims=True)
    xt = xt * jax.lax.rsqrt(var + EPS) * (nw * scale)
    x1 = xt[:RD // 2, :]
    x2 = xt[RD // 2:RD, :]
    xp = xt[RD:, :]
    r1 = x1 * ct[:RD // 2, :] - x2 * st[:RD // 2, :]
    r2 = x2 * ct[RD // 2:, :] + x1 * st[RD // 2:, :]
    return jnp.concatenate([r1, r2, xp], axis=0)


VROWS = HD + 8


def _attn_body(q_ref, k_ref, v_ref, o_ref, kb_ref, vb_ref):
    i = pl.program_id(1)

    @pl.when(i == 0)
    def _():
        kb_ref[...] = k_ref[0].T
        vb_ref[:HD, :] = v_ref[0]
        vb_ref[HD:HD + 1, :] = jnp.ones((1, S), _bf16)
        vb_ref[HD + 1:, :] = jnp.zeros((VROWS - HD - 1, S), _bf16)

    qblk = q_ref[0]
    qt = jnp.concatenate([qblk[h * HD:(h + 1) * HD, :] for h in range(GQ)],
                         axis=1)

    def tile(j, masked):
        kc = kb_ref[pl.ds(j * BK, BK), :]
        s = _dot(kc, qt, ((1,), (0,)))
        p = jnp.exp(s)
        if masked:
            kpos = j * BK + jax.lax.broadcasted_iota(jnp.int32, (BK, W), 0)
            qpos = i * BQ + (jax.lax.broadcasted_iota(jnp.int32, (BK, W), 1)
                             & (BQ - 1))
            p = jnp.where(kpos <= qpos, p, 0.0)
        return _dot(vb_ref[:, pl.ds(j * BK, BK)], p.astype(_bf16),
                    ((1,), (0,)))

    def pair(t, acc, masked):
        return acc + tile(2 * t, masked) + tile(2 * t + 1, masked)

    acc = jnp.zeros((VROWS, W), _f32)
    acc = jax.lax.fori_loop(0, i, lambda t, a: pair(t, a, False), acc)
    acc = pair(i, acc, True)
    o_ref[0] = acc[:HD, :] / acc[HD:HD + 1, :]


def _router_body(a0_ref, a1_ref, a2_ref, a3_ref, owb_ref, xt_ref, ln2_ref,
                 ghi_ref, glo_ref, hid_ref, flat_ref, comb_ref):
    h2 = xt_ref[...]
    for g, aref in enumerate((a0_ref, a1_ref, a2_ref, a3_ref)):
        h2 = h2 + _dot(owb_ref[g], aref[...].astype(_bf16), ((1,), (0,)))
    hid_ref[...] = h2
    var = jnp.mean(h2 * h2, axis=0, keepdims=True)
    flat = h2 * jax.lax.rsqrt(var + EPS) * ln2_ref[...]
    flat_ref[...] = flat
    hi = flat.astype(_bf16)
    lo = (flat - hi.astype(_f32)).astype(_bf16)
    logits = (_dot(ghi_ref[...], hi, ((1,), (0,)))
              + _dot(glo_ref[...], hi, ((1,), (0,)))
              + _dot(ghi_ref[...], lo, ((1,), (0,))))
    sc = jax.nn.sigmoid(logits)
    cols = [sc[e:e + 1, :] for e in range(E)]
    gsum = [cols[2 * g] + cols[2 * g + 1] for g in range(NG)]
    neg = jnp.float32(-jnp.inf)
    grank = []
    for gi in range(NG):
        r = jnp.zeros_like(gsum[gi])
        for gj in range(NG):
            if gj == gi:
                continue
            gt = gsum[gj] > gsum[gi]
            if gj < gi:
                gt = jnp.logical_or(gt, gsum[gj] == gsum[gi])
            r = r + gt.astype(_f32)
        grank.append(r)
    masked = [jnp.where(grank[e // 2] < TKG, cols[e], neg) for e in range(E)]
    w = []
    for ei in range(E):
        r = jnp.zeros_like(masked[ei])
        for ej in range(E):
            if ej == ei:
                continue
            gt = masked[ej] > masked[ei]
            if ej < ei:
                gt = jnp.logical_or(gt, masked[ej] == masked[ei])
            r = r + gt.astype(_f32)
        w.append(jnp.where(r < TOPK, cols[ei], 0.0))
    denom = w[0] + w[1] + w[2] + w[3] + w[4] + w[5] + w[6] + w[7] + 1e-20
    comb_ref[...] = jnp.concatenate(w, axis=0) / denom * RSF


def _moe_body(x_ref, h_ref, comb_ref, wg_ref, wu_ref, wd_ref,
              sg_ref, su_ref, sd_ref, out_ref, acc_ref):
    fc = pl.program_id(0)
    t = pl.program_id(1)
    xb = x_ref[...].astype(_bf16)
    tsl = pl.ds(t * BT, BT)

    @pl.when(fc == 0)
    def _():
        acc_ref[:, tsl] = h_ref[...]

    acc = acc_ref[:, tsl]
    acts = []
    for e in range(E):
        g = _dot(wg_ref[e], xb, ((1,), (0,)))
        u = _dot(wu_ref[e], xb, ((1,), (0,)))
        acts.append((g * jax.nn.sigmoid(g) * u).astype(_bf16))
    g = _dot(sg_ref[...], xb, ((1,), (0,)))
    u = _dot(su_ref[...], xb, ((1,), (0,)))
    acts.append((g * jax.nn.sigmoid(g) * u).astype(_bf16))
    for e in range(E):
        y = _dot(wd_ref[e], acts[e], ((1,), (0,)))
        acc = acc + y * comb_ref[e:e + 1, :]
    acc = acc + _dot(sd_ref[...], acts[E], ((1,), (0,)))
    acc_ref[:, tsl] = acc
    out_ref[...] = acc.T


def kernel(hidden_states, cos, sin, ln1_w, ln2_w, q_w, k_w, v_w, o_w,
           qn_w, kn_w, gate_w, ew_gate, ew_up, ew_down,
           sw_gate, sw_up, sw_down):
    x = hidden_states.reshape(S, D)
    cost = cos.reshape(S, RD).T
    sint = sin.reshape(S, RD).T

    wqkv = jnp.concatenate([q_w, k_w, v_w], axis=0).T.astype(_bf16)

    qkvt, xt = pl.pallas_call(
        _qkv_body,
        grid=(S // BT,),
        in_specs=[
            pl.BlockSpec((BT, D), lambda i: (i, 0)),
            pl.BlockSpec((D, QKVW), lambda i: (0, 0)),
            pl.BlockSpec((1, D), lambda i: (0, 0)),
            pl.BlockSpec((RD, BT), lambda i: (0, i)),
            pl.BlockSpec((RD, BT), lambda i: (0, i)),
            pl.BlockSpec((HD, 1), lambda i: (0, 0)),
            pl.BlockSpec((HD, 1), lambda i: (0, 0)),
        ],
        out_specs=[
            pl.BlockSpec((QKVW, BT), lambda i: (0, i)),
            pl.BlockSpec((D, BT), lambda i: (0, i)),
        ],
        out_shape=[
            jax.ShapeDtypeStruct((QKVW, S), _bf16),
            jax.ShapeDtypeStruct((D, S), _f32),
        ],
        compiler_params=pltpu.CompilerParams(
            dimension_semantics=("parallel",)),
    )(x, wqkv, ln1_w.reshape(1, D), cost, sint,
      qn_w.reshape(HD, 1), kn_w.reshape(HD, 1))

    q4 = qkvt[:NH * HD].reshape(NKV, GQ * HD, S)
    k3t = qkvt[NH * HD:NH * HD + NKV * HD].reshape(NKV, HD, S)
    v3t = qkvt[NH * HD + NKV * HD:].reshape(NKV, HD, S)

    aa = pl.pallas_call(
        _attn_body,
        grid=(NKV, S // BQ),
        in_specs=[
            pl.BlockSpec((1, GQ * HD, BQ), lambda kv, i: (kv, 0, i)),
            pl.BlockSpec((1, HD, S), lambda kv, i: (kv, 0, 0)),
            pl.BlockSpec((1, HD, S), lambda kv, i: (kv, 0, 0)),
        ],
        out_specs=pl.BlockSpec((1, HD, W), lambda kv, i: (kv, 0, i)),
        out_shape=jax.ShapeDtypeStruct((NKV, HD, (S // BQ) * W), _f32),
        scratch_shapes=[pltpu.VMEM((S, HD), _bf16),
                        pltpu.VMEM((VROWS, S), _bf16)],
        compiler_params=pltpu.CompilerParams(
            dimension_semantics=("parallel", "arbitrary")),
    )(q4, k3t, v3t)

    aa2 = aa.reshape(NKV * HD, (S // BQ) * W)

    owb = (o_w.reshape(D, NKV, GQ, HD).transpose(2, 0, 1, 3)
           .reshape(GQ, D, NKV * HD).astype(_bf16))
    ghi = gate_w.astype(_bf16)
    glo = (gate_w - ghi.astype(_f32)).astype(_bf16)

    nb = BQ // BT

    def band_spec(g):
        return pl.BlockSpec(
            (NKV * HD, BT),
            lambda t, g=g: (0, (t // nb) * (W // BT) + g * (BQ // BT)
                            + (t % nb)))

    hidt, flatt, combt = pl.pallas_call(
        _router_body,
        grid=(S // BT,),
        in_specs=[
            band_spec(0), band_spec(1), band_spec(2), band_spec(3),
            pl.BlockSpec((GQ, D, NKV * HD), lambda t: (0, 0, 0)),
            pl.BlockSpec((D, BT), lambda t: (0, t)),
            pl.BlockSpec((D, 1), lambda t: (0, 0)),
            pl.BlockSpec((E, D), lambda t: (0, 0)),
            pl.BlockSpec((E, D), lambda t: (0, 0)),
        ],
        out_specs=[
            pl.BlockSpec((D, BT), lambda t: (0, t)),
            pl.BlockSpec((D, BT), lambda t: (0, t)),
            pl.BlockSpec((E, BT), lambda t: (0, t)),
        ],
        out_shape=[
            jax.ShapeDtypeStruct((D, S), _f32),
            jax.ShapeDtypeStruct((D, S), _f32),
            jax.ShapeDtypeStruct((E, S), _f32),
        ],
        compiler_params=pltpu.CompilerParams(
            dimension_semantics=("parallel",)),
    )(aa2, aa2, aa2, aa2, owb, xt, ln2_w.reshape(D, 1), ghi, glo)

    wgb = ew_gate.astype(_bf16)
    wub = ew_up.astype(_bf16)
    wdb = ew_down.astype(_bf16)
    sgb = sw_gate.astype(_bf16)
    sub = sw_up.astype(_bf16)
    sdb = sw_down.astype(_bf16)

    out = pl.pallas_call(
        _moe_body,
        grid=(FF // FH, S // BT),
        in_specs=[
            pl.BlockSpec((D, BT), lambda fc, t: (0, t)),
            pl.BlockSpec((D, BT), lambda fc, t: (0, t)),
            pl.BlockSpec((E, BT), lambda fc, t: (0, t)),
            pl.BlockSpec((E, FH, D), lambda fc, t: (0, fc, 0)),
            pl.BlockSpec((E, FH, D), lambda fc, t: (0, fc, 0)),
            pl.BlockSpec((E, D, FH), lambda fc, t: (0, 0, fc)),
            pl.BlockSpec((FH, D), lambda fc, t: (fc, 0)),
            pl.BlockSpec((FH, D), lambda fc, t: (fc, 0)),
            pl.BlockSpec((D, FH), lambda fc, t: (0, fc)),
        ],
        out_specs=pl.BlockSpec((BT, D), lambda fc, t: (t, 0)),
        out_shape=jax.ShapeDtypeStruct((S, D), _f32),
        scratch_shapes=[pltpu.VMEM((D, S), _f32)],
        compiler_params=pltpu.CompilerParams(
            dimension_semantics=("arbitrary", "parallel")),
    )(flatt, hidt, combt, wgb, wub, wdb, sgb, sub, sdb)

    return out.reshape(1, S, D)
